# baseline (speedup 1.0000x reference)
import functools

import jax
import jax.numpy as jnp
from jax.experimental import pallas as pl
from jax.experimental.pallas import tpu as pltpu


def _assign_kernel(x_ref, means_ref, out_ref, *, n_clusters):
    b, l, d = x_ref.shape
    h = pl.program_id(0)
    xb = x_ref[...].reshape(b * l, d)
    m = means_ref[0]
    s = jax.lax.dot_general(m, xb, (((1,), (1,)), ((), ())),
                            precision=jax.lax.Precision.DEFAULT,
                            preferred_element_type=jnp.float32)
    mx = jnp.max(s, axis=0, keepdims=True)
    iota = jax.lax.broadcasted_iota(jnp.int32, s.shape, 0)
    idx = jnp.min(jnp.where(s == mx, iota, jnp.int32(2**30)),
                  axis=0, keepdims=True)
    codes = idx + h * n_clusters
    out_ref[...] = codes.reshape(b, l)


@jax.jit
def kernel(x, means):
    b, l, d = x.shape
    n_rounds, n_clusters, _ = means.shape

    out = pl.pallas_call(
        functools.partial(_assign_kernel, n_clusters=n_clusters),
        grid=(n_rounds,),
        in_specs=[
            pl.BlockSpec((b, l, d), lambda h: (0, 0, 0)),
            pl.BlockSpec((1, n_clusters, d), lambda h: (h, 0, 0)),
        ],
        out_specs=pl.BlockSpec((b, l), lambda h: (0, h)),
        out_shape=jax.ShapeDtypeStruct((b, n_rounds * l), jnp.int32),
    )(x, means)

    return out


# speedup vs baseline: 1.3532x; 1.3532x over previous
import functools

import jax
import jax.numpy as jnp
import numpy as np
from jax.experimental import pallas as pl
from jax.experimental.pallas import tpu as pltpu


def _argmax_rows_bitrev(s):
    val = s
    off = None
    k = 0
    while val.shape[0] > 1:
        half = val.shape[0] // 2
        a, bb = val[:half], val[half:]
        take_b = bb > a
        val = jnp.maximum(a, bb)
        if off is None:
            off = jnp.where(take_b, jnp.int32(1), jnp.int32(0))
        else:
            off = jnp.where(take_b, off[half:] | jnp.int32(1 << k), off[:half])
        k += 1
    return off


def _assign_kernel(xt_ref, means_ref, out_ref, *, n_rounds, n_clusters):
    xt = xt_ref[...]
    for h in range(n_rounds):
        m = means_ref[h]
        s = jax.lax.dot(m, xt, precision=jax.lax.Precision.DEFAULT,
                        preferred_element_type=jnp.float32)
        idx = _argmax_rows_bitrev(s)
        out_ref[0, h:h + 1, :] = idx + jnp.int32(h * n_clusters)


def _bitrev_perm(n):
    bits = int(np.log2(n))
    i = np.arange(n)
    r = np.zeros_like(i)
    for k in range(bits):
        r |= ((i >> k) & 1) << (bits - 1 - k)
    return r


@jax.jit
def kernel(x, means):
    b, l, d = x.shape
    n_rounds, n_clusters, _ = means.shape
    n_tokens = b * l

    block_r = 1024
    nb_per_b = l // block_r
    grid = (n_tokens // block_r,)

    xt = x.reshape(n_tokens, d).T
    means_br = means[:, _bitrev_perm(n_clusters), :]

    out = pl.pallas_call(
        functools.partial(_assign_kernel, n_rounds=n_rounds,
                          n_clusters=n_clusters),
        grid=grid,
        in_specs=[
            pl.BlockSpec((d, block_r), lambda i: (0, i)),
            pl.BlockSpec((n_rounds, n_clusters, d), lambda i: (0, 0, 0)),
        ],
        out_specs=pl.BlockSpec((1, n_rounds, block_r),
                               lambda i: (i // nb_per_b, 0, i % nb_per_b)),
        out_shape=jax.ShapeDtypeStruct((b, n_rounds, l), jnp.int32),
    )(xt, means_br)

    return out.reshape(b, n_rounds * l)


# speedup vs baseline: 1.3755x; 1.0165x over previous
import functools

import jax
import jax.numpy as jnp
import numpy as np
from jax.experimental import pallas as pl
from jax.experimental.pallas import tpu as pltpu


def _argmax_rows_bitrev(s):
    val = s
    off = None
    k = 0
    while val.shape[0] > 1:
        half = val.shape[0] // 2
        a, bb = val[:half], val[half:]
        take_b = bb > a
        val = jnp.maximum(a, bb)
        if off is None:
            off = jnp.where(take_b, jnp.int32(1), jnp.int32(0))
        else:
            off = jnp.where(take_b, off[half:] | jnp.int32(1 << k), off[:half])
        k += 1
    return off


def _assign_kernel(xt_ref, means_ref, out_ref, *, n_rounds, n_clusters):
    xt = xt_ref[...]
    for h in range(n_rounds):
        m = means_ref[h]
        s = jax.lax.dot(m, xt, precision=jax.lax.Precision.DEFAULT,
                        preferred_element_type=jnp.float32)
        idx = _argmax_rows_bitrev(s)
        out_ref[0, h:h + 1, :] = idx + jnp.int32(h * n_clusters)


def _bitrev_perm(n):
    bits = int(np.log2(n))
    i = np.arange(n)
    r = np.zeros_like(i)
    for k in range(bits):
        r |= ((i >> k) & 1) << (bits - 1 - k)
    return r


@jax.jit
def kernel(x, means):
    b, l, d = x.shape
    n_rounds, n_clusters, _ = means.shape
    n_tokens = b * l

    block_r = 2048
    nb_per_b = l // block_r
    grid = (n_tokens // block_r,)

    xt = x.reshape(n_tokens, d).T
    means_br = means[:, _bitrev_perm(n_clusters), :]

    out = pl.pallas_call(
        functools.partial(_assign_kernel, n_rounds=n_rounds,
                          n_clusters=n_clusters),
        grid=grid,
        in_specs=[
            pl.BlockSpec((d, block_r), lambda i: (0, i)),
            pl.BlockSpec((n_rounds, n_clusters, d), lambda i: (0, 0, 0)),
        ],
        out_specs=pl.BlockSpec((1, n_rounds, block_r),
                               lambda i: (i // nb_per_b, 0, i % nb_per_b)),
        out_shape=jax.ShapeDtypeStruct((b, n_rounds, l), jnp.int32),
    )(xt, means_br)

    return out.reshape(b, n_rounds * l)
